# baseline (speedup 1.0000x reference)
import functools

import jax
import jax.numpy as jnp
from jax import lax
from jax.experimental import pallas as pl
from jax.experimental.pallas import tpu as pltpu
from jax.experimental.pallas import tpu_sc as plsc

VOCAB = 1000000
NUM_PROMPT = 128
D = 32
B = 4096
H = 200

NC, NS, L = 2, 16, 16
NW = NC * NS
BW = B // NW
CH = 8
NCHUNK = H // CH
BLK = CH * BW
GPER = 128
NG = BLK // GPER


def _body(orig_hbm, new_t_hbm, idx_t_hbm, out_hbm,
          newt_v, idxb_v, idxf_v, safe_v, rows_v, trans_v, sem):
    wid = lax.axis_index("s") * NC + lax.axis_index("c")
    b0 = wid * BW

    pltpu.sync_copy(new_t_hbm, newt_v)

    def chunk(q, carry):
        h0 = q * CH
        pltpu.sync_copy(idx_t_hbm.at[pl.ds(h0, CH), pl.ds(b0, BW)], idxb_v)

        for r in range(CH):
            for c in range(BW // L):
                v = idxb_v[r, pl.ds(c * L, L)]
                idxf_v[pl.ds(r * BW + c * L, L)] = v
                safe_v[pl.ds(r * BW + c * L, L)] = jnp.minimum(v, VOCAB - 1)

        descs = [
            pltpu.async_copy(
                orig_hbm.at[safe_v.at[pl.ds(j * GPER, GPER)]],
                rows_v.at[pl.ds(j * GPER, GPER)],
                sem,
            )
            for j in range(NG)
        ]
        for dsc in descs:
            dsc.wait()

        def fix(t, c):
            v = idxf_v[pl.ds(t * L, L)]
            cnt = plsc.all_reduce_population_count(v >= VOCAB)

            @pl.when(cnt[0] > 0)
            def _():
                mask = v >= VOCAB
                pidx = jnp.maximum(v - VOCAB, 0)
                rowid = t * L + lax.iota(jnp.int32, L)
                for dd in range(D):
                    dvec = jnp.full((L,), dd, jnp.int32)
                    vals = plsc.load_gather(newt_v, [dvec, pidx])
                    plsc.store_scatter(rows_v, [rowid, dvec], vals, mask=mask)
            return c
        lax.fori_loop(0, BLK // L, fix, 0)

        lanes = lax.iota(jnp.int32, L)

        def tpose(hc, c):
            h = hc // D
            dd = hc % D
            dvec = jnp.full((L,), dd, jnp.int32)
            for g in range(BW // L):
                rowid = h * BW + g * L + lanes
                vals = plsc.load_gather(rows_v, [rowid, dvec])
                trans_v[h, dd, pl.ds(g * L, L)] = vals
            return c
        lax.fori_loop(0, CH * D, tpose, 0)

        pltpu.sync_copy(trans_v,
                        out_hbm.at[pl.ds(h0, CH), :, pl.ds(b0, BW)])
        return carry

    lax.fori_loop(0, NCHUNK, chunk, 0)


@functools.partial(jax.jit, static_argnames=())
def _lookup(orig_weight, new_t, idx_t):
    mesh = plsc.VectorSubcoreMesh(core_axis_name="c", subcore_axis_name="s")
    f = pl.kernel(
        _body,
        out_type=jax.ShapeDtypeStruct((H, D, B), jnp.float32),
        mesh=mesh,
        scratch_types=[
            pltpu.VMEM((D, NUM_PROMPT), jnp.float32),
            pltpu.VMEM((CH, BW), jnp.int32),
            pltpu.VMEM((BLK,), jnp.int32),
            pltpu.VMEM((BLK,), jnp.int32),
            pltpu.VMEM((BLK, D), jnp.float32),
            pltpu.VMEM((CH, D, BW), jnp.float32),
            pltpu.SemaphoreType.DMA,
        ],
        compiler_params=pltpu.CompilerParams(
            needs_layout_passes=False, use_tc_tiling_on_sc=False),
    )
    return f(orig_weight, new_t, idx_t)


def kernel(orig_weight, new_weight, input):
    idx_t = input.astype(jnp.int32).T
    new_t = new_weight.T
    out = _lookup(orig_weight, new_t, idx_t)
    return out.transpose(2, 0, 1)


# speedup vs baseline: 1.0009x; 1.0009x over previous
import functools

import jax
import jax.numpy as jnp
from jax import lax
from jax.experimental import pallas as pl
from jax.experimental.pallas import tpu as pltpu
from jax.experimental.pallas import tpu_sc as plsc

VOCAB = 1000000
NUM_PROMPT = 128
D = 32
B = 4096
H = 200

NC, NS, L = 2, 16, 16
NWB = 4
NWH = 8
BW = B // NWB
HW = H // NWH
GPER = 128
NG = BW // GPER


def _body(orig_hbm, new_t_hbm, idx_t_hbm, out_hbm,
          newt_v, idxf_v, safe_v, rows_v, trans_v, sem):
    wid = lax.axis_index("s") * NC + lax.axis_index("c")
    wh = wid // NWB
    wb = wid % NWB
    b0 = wb * BW
    h0 = wh * HW

    pltpu.sync_copy(new_t_hbm, newt_v)

    lanes = lax.iota(jnp.int32, L)

    def hrow(q, carry):
        h = h0 + q
        pltpu.sync_copy(idx_t_hbm.at[h, pl.ds(b0, BW)], idxf_v)

        for c in range(BW // L):
            v = idxf_v[pl.ds(c * L, L)]
            safe_v[pl.ds(c * L, L)] = jnp.minimum(v, VOCAB - 1)

        descs = [
            pltpu.async_copy(
                orig_hbm.at[safe_v.at[pl.ds(j * GPER, GPER)]],
                rows_v.at[pl.ds(j * GPER, GPER)],
                sem,
            )
            for j in range(NG)
        ]
        for dsc in descs:
            dsc.wait()

        def fix(t, c):
            v = idxf_v[pl.ds(t * L, L)]
            cnt = plsc.all_reduce_population_count(v >= VOCAB)

            @pl.when(cnt[0] > 0)
            def _():
                mask = v >= VOCAB
                pidx = jnp.maximum(v - VOCAB, 0)
                rowid = t * L + lanes
                for dd in range(D):
                    dvec = jnp.full((L,), dd, jnp.int32)
                    vals = plsc.load_gather(newt_v, [dvec, pidx])
                    plsc.store_scatter(rows_v, [rowid, dvec], vals, mask=mask)
            return c
        lax.fori_loop(0, BW // L, fix, 0)

        def tpose(g, c):
            rowid = g * L + lanes
            for dd in range(D):
                dvec = jnp.full((L,), dd, jnp.int32)
                vals = plsc.load_gather(rows_v, [rowid, dvec])
                trans_v[dd, pl.ds(g * L, L)] = vals
            return c
        lax.fori_loop(0, BW // L, tpose, 0)

        pltpu.sync_copy(trans_v, out_hbm.at[h, :, pl.ds(b0, BW)])
        return carry

    lax.fori_loop(0, HW, hrow, 0)


@functools.partial(jax.jit, static_argnames=())
def _lookup(orig_weight, new_t, idx_t):
    mesh = plsc.VectorSubcoreMesh(core_axis_name="c", subcore_axis_name="s")
    f = pl.kernel(
        _body,
        out_type=jax.ShapeDtypeStruct((H, D, B), jnp.float32),
        mesh=mesh,
        scratch_types=[
            pltpu.VMEM((D, NUM_PROMPT), jnp.float32),
            pltpu.VMEM((BW,), jnp.int32),
            pltpu.VMEM((BW,), jnp.int32),
            pltpu.VMEM((BW, D), jnp.float32),
            pltpu.VMEM((D, BW), jnp.float32),
            pltpu.SemaphoreType.DMA,
        ],
        compiler_params=pltpu.CompilerParams(
            needs_layout_passes=False, use_tc_tiling_on_sc=False),
    )
    return f(orig_weight, new_t, idx_t)


def kernel(orig_weight, new_weight, input):
    idx_t = input.astype(jnp.int32).T
    new_t = new_weight.T
    out = _lookup(orig_weight, new_t, idx_t)
    return out.transpose(2, 0, 1)
